# Initial kernel scaffold; baseline (speedup 1.0000x reference)
import jax
import jax.numpy as jnp
from jax.experimental import pallas as pl


def kernel(x, edge_index, edge_attr, params):
    raise NotImplementedError("write your pallas kernel here")


# speedup vs baseline: 13.8071x; 13.8071x over previous
import jax
import jax.numpy as jnp
from jax import lax
from jax.experimental import pallas as pl

E = 1000
EP = 1024
N = 10
NP = 16


def _dotd(a, b):
    return jnp.dot(a.astype(jnp.bfloat16), b.astype(jnp.bfloat16),
                   preferred_element_type=jnp.float32)


def _split3(x):
    hi = x.astype(jnp.bfloat16)
    r = x - hi.astype(jnp.float32)
    mid = r.astype(jnp.bfloat16)
    lo = (r - mid.astype(jnp.float32)).astype(jnp.bfloat16)
    return hi, mid, lo


def _main_body(x_ref, src_ref, dst_ref, ea_ref, *rest):
    layers = [tuple(rest[6 * i + j][...] for j in range(6)) for i in range(6)]
    node_W, edge_W, reg_W, cls_W, node_b, edge_b, reg_b, cls_b = (
        r[...] for r in rest[36:44])
    out_ref = rest[44]

    iota = lax.broadcasted_iota(jnp.int32, (1, NP), 1)
    PM = src_ref[...] == iota
    QM = dst_ref[...] == iota
    Qb = jnp.where(QM, 1.0, 0.0).astype(jnp.bfloat16)

    h = x_ref[...]
    e = ea_ref[...]
    for k in range(6):
        W, W_e, att_src, att_dst, att_edge, b = layers[k]
        g = _dotd(h, W)
        e = _dotd(e, W_e)
        a_src = jnp.sum(g * att_src, axis=1).reshape(1, NP)
        a_dst = jnp.sum(g * att_dst, axis=1).reshape(1, NP)
        a_edge = jnp.sum(e * att_edge, axis=1, keepdims=True)
        t = (jnp.sum(jnp.where(PM, a_src, 0.0), axis=1, keepdims=True)
             + jnp.sum(jnp.where(QM, a_dst, 0.0), axis=1, keepdims=True)
             + a_edge)
        t = jnp.where(t >= 0.0, t, 0.2 * t)
        amax = jnp.max(jnp.where(QM, t, -jnp.inf), axis=0, keepdims=True)
        amax = jnp.where(jnp.isfinite(amax), amax, 0.0)
        ex = jnp.exp(t - jnp.sum(jnp.where(QM, amax, 0.0),
                                 axis=1, keepdims=True))
        exm = jnp.where(PM, ex, 0.0)
        hi, mid, lo = _split3(exm)
        S = (jnp.dot(Qb.T, hi, preferred_element_type=jnp.float32)
             + jnp.dot(Qb.T, mid, preferred_element_type=jnp.float32)
             + jnp.dot(Qb.T, lo, preferred_element_type=jnp.float32))
        denom = jnp.sum(jnp.where(QM, ex, 0.0), axis=0).reshape(NP, 1)
        A = S / (denom + 1e-16)
        h = b * jnp.ones((NP, 1), jnp.float32)
        for m in range(N):
            h = h + A[:, m:m + 1] * g[m:m + 1, :]

    xn = _dotd(node_W, h) + node_b
    en = _dotd(edge_W, e) + edge_b
    s = xn + en
    v = _dotd(jnp.tanh(s), reg_W.reshape(-1, 1)) + reg_b
    c = _dotd(jnp.maximum(s, 0.0), cls_W.reshape(-1, 1)) + cls_b
    c = 1.0 / (1.0 + jnp.exp(-c))
    out_ref[...] = jnp.concatenate([v, c], axis=1)


def kernel(x, edge_index, edge_attr, params):
    src = edge_index[0].astype(jnp.int32)
    dst = edge_index[1].astype(jnp.int32)
    pad = jnp.full((EP - E,), -1, jnp.int32)
    src_col = jnp.concatenate([src, pad]).reshape(EP, 1)
    dst_col = jnp.concatenate([dst, pad]).reshape(EP, 1)

    xp = jnp.zeros((NP, x.shape[1]), jnp.float32).at[:N].set(x)
    eap = jnp.zeros((EP, edge_attr.shape[1]), jnp.float32).at[:E].set(edge_attr)

    pvals = []
    for n in ('c1', 'c2', 'c3', 'c4', 'c5', 'c6'):
        p = params[n]
        pvals += [p['W'], p['W_e'], p['att_src'][None, :],
                  p['att_dst'][None, :], p['att_edge'][None, :], p['b'][None, :]]
    node_W = jnp.zeros((1, NP), jnp.float32).at[0, :N].set(params['node_W'][:, 0])
    edge_W = jnp.zeros((1, EP), jnp.float32).at[0, :E].set(params['edge_W'][:, 0])
    pvals += [node_W, edge_W, params['reg_W'].T, params['cls_W'].T,
              params['node_b'][None, :], params['edge_b'][None, :],
              params['reg_b'][None, :], params['cls_b'][None, :]]

    out = pl.pallas_call(
        _main_body,
        out_shape=jax.ShapeDtypeStruct((1, 2), jnp.float32),
    )(xp, src_col, dst_col, eap, *pvals)
    return (out[0, 0].reshape(1), out[0, 1].reshape(1))
